# baseline (speedup 1.0000x reference)
import functools

import jax
import jax.numpy as jnp
from jax import lax
from jax.experimental import pallas as pl
from jax.experimental.pallas import tpu as pltpu
from jax.experimental.pallas import tpu_sc as plsc

N_BITS = 24
N = 262144
_ROWS = 2048

NCORES = 2
NSUB = 16
KPT = N // NSUB
HIST = 1 << 19
NRIT = 16
TRB = 1 << 15
NREG = 512
SEGT = NREG // NSUB
STAGE = 330368
FPAD = 256
FSIZE = N + FPAD
ZSTR = FSIZE // NSUB


def _iota16():
    return lax.iota(jnp.int32, 16)


def _scalar(x):
    if getattr(x, "ndim", 0):
        return x[0]
    return x


def _pack_body(probs_ref, u_ref, keys_ref):
    u = u_ref[...]
    probs = probs_ref[...]
    bits = (u < probs[None, :]).astype(jnp.int32)
    powers = (1 << lax.broadcasted_iota(jnp.int32, (1, N_BITS), 1))
    keys_ref[...] = jnp.sum(bits * powers, axis=1)


def _pack_keys(probs, u):
    return pl.pallas_call(
        _pack_body,
        grid=(N // _ROWS,),
        in_specs=[
            pl.BlockSpec((N_BITS,), lambda i: (0,)),
            pl.BlockSpec((_ROWS, N_BITS), lambda i: (i, 0)),
        ],
        out_specs=pl.BlockSpec((_ROWS,), lambda i: (i,)),
        out_shape=jax.ShapeDtypeStruct((N,), jnp.int32),
    )(probs, u)


def _bits_body(keys_ref, bits_ref):
    k = keys_ref[...]
    j = lax.broadcasted_iota(jnp.int32, (1, N_BITS), 1)
    bits_ref[...] = ((k[:, None] >> j) & 1).astype(jnp.int8)


def _expand_bits(fkeys):
    return pl.pallas_call(
        _bits_body,
        grid=(N // _ROWS,),
        in_specs=[pl.BlockSpec((_ROWS,), lambda i: (i,))],
        out_specs=pl.BlockSpec((_ROWS, N_BITS), lambda i: (i, 0)),
        out_shape=jax.ShapeDtypeStruct((N, N_BITS), jnp.int8),
    )(fkeys)


def _a_body(keys_hbm, stage_k, stage_c, nnz_hbm, pv_hbm,
            keys_v, idxflat, idx2d, ones_row, rowtmp, zeros_v,
            chunkbuf, outk_v, outc_v, coarse_all, pv_v, nnz_acc,
            hist_sh, coarse_sh, semS, semF):
    c_ax = lax.axis_index("c")
    s_ax = lax.axis_index("s")
    i16 = _iota16()

    def _init(i, _):
        zeros_v[pl.ds(i * 16, 16)] = jnp.zeros((16,), jnp.int32)
        return 0
    lax.fori_loop(0, KPT // 16, _init, 0)
    for u in range(8):
        ones_row[pl.ds(u * 16, 16)] = jnp.ones((16,), jnp.int32)

    pltpu.sync_copy(keys_hbm.at[pl.ds(pl.multiple_of(s_ax * KPT, 128), KPT)], keys_v)

    @pl.when(s_ax == 0)
    def _():
        pltpu.sync_copy(zeros_v.at[pl.ds(0, NREG)], coarse_sh)
    plsc.subcore_barrier()

    def _coarse_row(r, _):
        for u in range(8):
            k = keys_v[pl.ds(r * 128 + u * 16, 16)]
            idx2d[r, pl.ds(u * 16, 16)] = k >> 15
        pltpu.sync_copy(ones_row, coarse_sh.at[idx2d.at[r]], add=True)
        return 0
    lax.fori_loop(0, 128, _coarse_row, 0)

    def _drain(n, sem):
        def _w(i, _):
            pltpu.make_async_copy(keys_hbm.at[pl.ds(0, 128)], rowtmp, sem).wait()
            return 0
        lax.fori_loop(0, n, _w, 0)
    plsc.subcore_barrier()
    def _dly(i, _):
        idxflat[pl.ds(i * 16, 16)] = jnp.zeros((16,), jnp.int32)
        return 0
    lax.fori_loop(0, 1024, _dly, 0)
    plsc.subcore_barrier()

    pltpu.sync_copy(coarse_sh, coarse_all)
    carry = jnp.int32(0)
    for t in range(NREG // 16):
        cvec = coarse_all[pl.ds(t * 16, 16)]
        csum = plsc.cumsum(cvec)
        excl = csum - cvec + carry
        carry = carry + _scalar(csum[15:16])
        rt = i16 + t * 16
        pv_v[pl.ds(t * 16, 16)] = ((excl + 127) >> 7 << 7) + 128 * rt
    @pl.when((s_ax == 0) & (c_ax == 0))
    def _():
        pltpu.sync_copy(pv_v, pv_hbm)

    def _compress(rr):
        base = (c_ax * NRIT + rr) * HIST

        def _grp(g, pos):
            for u in range(8):
                k = keys_v[pl.ds(g * 128 + u * 16, 16)]
                d = k - base
                m = d.astype(jnp.uint32) < jnp.uint32(HIST)
                plsc.store_compressed(idxflat.at[pl.ds(pos, 16)], d, mask=m)
                pos = pos + _scalar(plsc.all_reduce_population_count(m))
            return pos
        pos = lax.fori_loop(0, 128, _grp, jnp.int32(0))
        for u in range(8):
            idxflat[pl.ds(pos + u * 16, 16)] = HIST + u * 16 + i16
        return pos

    for q in range(TRB // KPT):
        pltpu.sync_copy(zeros_v, hist_sh.at[pl.ds(pl.multiple_of(s_ax * TRB + q * KPT, 128), KPT)])
    pos0 = _compress(jnp.int32(0))
    plsc.subcore_barrier()

    def _riter(riter, pos_cur):
        range_id = c_ax * NRIT + riter

        nrows = (pos_cur >> 7) + 1
        def _row(r, _):
            for u in range(8):
                idx2d[r, pl.ds(u * 16, 16)] = idxflat[pl.ds(r * 128 + u * 16, 16)]
            pltpu.sync_copy(ones_row, hist_sh.at[idx2d.at[r]], add=True)
            return 0
        lax.fori_loop(0, nrows, _row, 0)
        plsc.subcore_barrier()

        pos_next = _compress(riter + 1)
        plsc.subcore_barrier()

        rtv = jnp.where(i16 == s_ax, pv_v[pl.ds(range_id * 16, 16)], 0)
        pbase = _scalar(plsc.cumsum(rtv)[15:16])
        kbase0 = range_id * HIST + s_ax * TRB

        def _chunk(ch, carry2):
            cpos, gpos = carry2
            pltpu.sync_copy(hist_sh.at[pl.ds(pl.multiple_of(s_ax * TRB + ch * 4096, 128), 4096)], chunkbuf)
            pltpu.sync_copy(zeros_v.at[pl.ds(0, 4096)], hist_sh.at[pl.ds(pl.multiple_of(s_ax * TRB + ch * 4096, 128), 4096)])

            def _vb(vb, cp):
                for u in range(8):
                    off = vb * 128 + u * 16
                    cnt = chunkbuf[pl.ds(off, 16)]
                    m = cnt > 0
                    kv = kbase0 + ch * 4096 + off + i16
                    plsc.store_compressed(outk_v.at[pl.ds(cp, 16)], kv, mask=m)
                    plsc.store_compressed(outc_v.at[pl.ds(cp, 16)], cnt, mask=m)
                    cp = cp + _scalar(plsc.all_reduce_population_count(m))
                return cp
            cpos = lax.fori_loop(0, 32, _vb, cpos)

            nfull = cpos >> 7
            def _flush(f, _):
                dst = pl.multiple_of(pbase + gpos + f * 128, 128)
                pltpu.async_copy(outk_v.at[pl.ds(f * 128, 128)],
                                 stage_k.at[pl.ds(dst, 128)], semF)
                pltpu.async_copy(outc_v.at[pl.ds(f * 128, 128)],
                                 stage_c.at[pl.ds(dst, 128)], semF)
                return 0
            lax.fori_loop(0, nfull, _flush, 0)
            _drain(2 * nfull, semF)
            tail = cpos & 127
            for u in range(8):
                tk = outk_v[pl.ds(nfull * 128 + u * 16, 16)]
                tc = outc_v[pl.ds(nfull * 128 + u * 16, 16)]
                outk_v[pl.ds(u * 16, 16)] = tk
                outc_v[pl.ds(u * 16, 16)] = tc
            return (tail, gpos + nfull * 128)

        cpos, gpos = lax.fori_loop(0, TRB // 4096, _chunk, (jnp.int32(0), jnp.int32(0)))

        nlast = (cpos + 127) >> 7
        def _flast(f, _):
            dst = pl.multiple_of(pbase + gpos + f * 128, 128)
            pltpu.async_copy(outk_v.at[pl.ds(f * 128, 128)],
                             stage_k.at[pl.ds(dst, 128)], semF)
            pltpu.async_copy(outc_v.at[pl.ds(f * 128, 128)],
                             stage_c.at[pl.ds(dst, 128)], semF)
            return 0
        lax.fori_loop(0, nlast, _flast, 0)
        _drain(2 * nlast, semF)

        nnz = gpos + cpos
        nnz_acc[...] = jnp.where(i16 == riter, nnz, nnz_acc[...])

        plsc.subcore_barrier()
        return pos_next

    lax.fori_loop(0, NRIT, _riter, pos0)
    pltpu.sync_copy(nnz_acc, nnz_hbm.at[pl.ds(pl.multiple_of((c_ax * 16 + s_ax) * 16, 16), 16)])


def _run_a(keys):
    mesh = plsc.VectorSubcoreMesh(core_axis_name="c", subcore_axis_name="s")
    f = pl.kernel(
        _a_body,
        out_type=[
            jax.ShapeDtypeStruct((STAGE,), jnp.int32),
            jax.ShapeDtypeStruct((STAGE,), jnp.int32),
            jax.ShapeDtypeStruct((NCORES * NSUB * 16,), jnp.int32),
            jax.ShapeDtypeStruct((NREG,), jnp.int32),
        ],
        mesh=mesh,
        scratch_types=[
            pltpu.VMEM((KPT,), jnp.int32),
            pltpu.VMEM((KPT + 144,), jnp.int32),
            pltpu.VMEM((129, 128), jnp.int32),
            pltpu.VMEM((128,), jnp.int32),
            pltpu.VMEM((128,), jnp.int32),
            pltpu.VMEM((KPT,), jnp.int32),
            pltpu.VMEM((4096,), jnp.int32),
            pltpu.VMEM((4224,), jnp.int32),
            pltpu.VMEM((4224,), jnp.int32),
            pltpu.VMEM((NREG,), jnp.int32),
            pltpu.VMEM((NREG,), jnp.int32),
            pltpu.VMEM((16,), jnp.int32),
            pltpu.VMEM_SHARED((HIST + 128,), jnp.int32),
            pltpu.VMEM_SHARED((NREG,), jnp.int32),
            pltpu.SemaphoreType.DMA,
            pltpu.SemaphoreType.DMA,
        ],
        compiler_params=pltpu.CompilerParams(needs_layout_passes=False),
    )
    return f(keys)


def _b_body(stage_k, stage_c, nnz_hbm, pv_hbm, fkeys, fcnts,
            zeros_v, nnzb, pv_v, qv_v, bk_v, bc_v, idx2d, rowtmp, semK):
    s_ax = lax.axis_index("s")
    i16 = _iota16()

    def _init(i, _):
        zeros_v[pl.ds(i * 16, 16)] = jnp.zeros((16,), jnp.int32)
        return 0
    lax.fori_loop(0, ZSTR // 16, _init, 0)

    pltpu.sync_copy(zeros_v, fkeys.at[pl.ds(pl.multiple_of(s_ax * ZSTR, 16), ZSTR)])
    pltpu.sync_copy(zeros_v, fcnts.at[pl.ds(pl.multiple_of(s_ax * ZSTR, 16), ZSTR)])

    pltpu.sync_copy(nnz_hbm, nnzb)
    pltpu.sync_copy(pv_hbm, pv_v)

    carry = jnp.int32(0)
    for t in range(NREG // 16):
        src_idx = (t >> 4) * 256 + i16 * 16 + (t & 15)
        nv = plsc.load_gather(nnzb, [src_idx])
        csum = plsc.cumsum(nv)
        qv_v[pl.ds(t * 16, 16)] = csum - nv + carry
        carry = carry + _scalar(csum[15:16])

    plsc.subcore_barrier()

    def _lane(vec, lane):
        return _scalar(plsc.cumsum(jnp.where(i16 == lane, vec, 0))[15:16])

    def _seg(job, _):
        rt = s_ax * SEGT + job
        lane = rt & 15
        q0 = _lane(qv_v[pl.ds(pl.multiple_of((rt >> 4) * 16, 16), 16)], lane)
        p0 = _lane(pv_v[pl.ds(pl.multiple_of((rt >> 4) * 16, 16), 16)], lane)
        nbase = ((rt >> 8) * 16 + lane) * 16
        nz = _lane(nnzb[pl.ds(pl.multiple_of(nbase, 16), 16)], (rt >> 4) & 15)

        nch = (nz + 2047) >> 11
        def _chunk(ch, _):
            pltpu.sync_copy(stage_k.at[pl.ds(pl.multiple_of(p0 + ch * 2048, 128), 2048)], bk_v)
            pltpu.sync_copy(stage_c.at[pl.ds(pl.multiple_of(p0 + ch * 2048, 128), 2048)], bc_v)
            rem = nz - ch * 2048
            for g in range(128):
                lid = g * 16 + i16
                m = lid < rem
                oidx = jnp.where(m, q0 + ch * 2048 + lid, N + (lid & 127))
                idx2d[g >> 3, pl.ds((g & 7) * 16, 16)] = oidx
            for r in range(16):
                pltpu.sync_copy(bk_v.at[pl.ds(r * 128, 128)],
                                 fkeys.at[idx2d.at[r]])
                pltpu.sync_copy(bc_v.at[pl.ds(r * 128, 128)],
                                 fcnts.at[idx2d.at[r]])
            return 0
        lax.fori_loop(0, nch, _chunk, 0)
        return 0
    lax.fori_loop(0, SEGT, _seg, 0)


def _run_b(stage_k, stage_c, nnz, pv):
    mesh = plsc.VectorSubcoreMesh(core_axis_name="c", subcore_axis_name="s",
                                  num_cores=1)
    f = pl.kernel(
        _b_body,
        out_type=[
            jax.ShapeDtypeStruct((FSIZE,), jnp.int32),
            jax.ShapeDtypeStruct((FSIZE,), jnp.int32),
        ],
        mesh=mesh,
        scratch_types=[
            pltpu.VMEM((ZSTR,), jnp.int32),
            pltpu.VMEM((NCORES * NSUB * 16,), jnp.int32),
            pltpu.VMEM((NREG,), jnp.int32),
            pltpu.VMEM((NREG,), jnp.int32),
            pltpu.VMEM((2048,), jnp.int32),
            pltpu.VMEM((2048,), jnp.int32),
            pltpu.VMEM((16, 128), jnp.int32),
            pltpu.VMEM((128,), jnp.int32),
            pltpu.SemaphoreType.DMA,
        ],
        compiler_params=pltpu.CompilerParams(needs_layout_passes=False),
    )
    return f(stage_k, stage_c, nnz, pv)


def kernel(kernel, u, num_samples):
    probs = jax.nn.sigmoid(2.0 * kernel)
    keys = _pack_keys(probs, u)
    stage_k, stage_c, nnz, pv = _run_a(keys)
    fkeys, fcnts = _run_b(stage_k, stage_c, nnz, pv)
    counts = fcnts[:N]
    bits = _expand_bits(fkeys[:N])
    return bits, counts
